# baseline (speedup 1.0000x reference)
import jax
import jax.numpy as jnp
from jax.experimental import pallas as pl
from jax.experimental.pallas import tpu as pltpu

_D = 1024
_E = 8
_H = 2048
_T = 2048

_TILE = 256
_NTILES = (2 * _T + _E * (_TILE - 1) + _TILE - 1) // _TILE
_NPAD = _NTILES * _TILE


def _routing_kernel(x_ref, wr_ref, counts_ref, metar_ref,
                    eot_ref, live_ref):
    logits = jax.lax.dot_general(
        x_ref[...].astype(jnp.bfloat16), wr_ref[...].astype(jnp.bfloat16),
        (((1,), (0,)), ((), ())),
        preferred_element_type=jnp.float32)
    t, e = logits.shape
    col = jax.lax.broadcasted_iota(jnp.int32, (t, e), 1)

    v1 = jnp.max(logits, axis=1, keepdims=True)
    i1 = jnp.min(jnp.where(logits == v1, col, e), axis=1, keepdims=True)
    masked = jnp.where(col == i1, -jnp.inf, logits)
    v2 = jnp.max(masked, axis=1, keepdims=True)
    i2 = jnp.min(jnp.where(masked == v2, col, e), axis=1, keepdims=True)

    e2 = jnp.exp(v2 - v1)
    w1 = 1.0 / (1.0 + e2)
    w2 = 1.0 - w1

    sel1 = (col == i1)
    sel2 = (col == i2)
    onehot = jnp.where(sel1 | sel2, 1.0, 0.0)
    counts_ref[...] = jnp.sum(onehot, axis=0, keepdims=True)

    ri = jax.lax.broadcasted_iota(jnp.int32, (t, t), 0)
    ci = jax.lax.broadcasted_iota(jnp.int32, (t, t), 1)
    tri = jnp.where(ci < ri, 1.0, 0.0).astype(jnp.bfloat16)
    ranks = jax.lax.dot_general(
        tri, onehot.astype(jnp.bfloat16), (((1,), (0,)), ((), ())),
        preferred_element_type=jnp.float32)

    ones_col = jnp.full((t, 1), 1.0, dtype=jnp.bfloat16)
    counts_col = jax.lax.dot_general(
        onehot.astype(jnp.bfloat16), ones_col, (((0,), (0,)), ((), ())),
        preferred_element_type=jnp.float32)
    cpad_col = ((counts_col.astype(jnp.int32) + _TILE - 1)
                // _TILE * _TILE).astype(jnp.float32)

    er = jax.lax.broadcasted_iota(jnp.int32, (e, e), 0)
    ec = jax.lax.broadcasted_iota(jnp.int32, (e, e), 1)
    low = jnp.where(ec < er, 1.0, 0.0).astype(jnp.bfloat16)
    pad_off_col = jax.lax.dot_general(
        low, cpad_col.astype(jnp.bfloat16), (((1,), (0,)), ((), ())),
        preferred_element_type=jnp.float32)
    pad_end_col = pad_off_col + cpad_col

    ntiles = eot_ref.shape[1]
    tcol = (jax.lax.broadcasted_iota(jnp.int32, (e, ntiles), 1)
            * _TILE).astype(jnp.float32)
    eot = jnp.sum(jnp.where(pad_end_col <= tcol, 1.0, 0.0),
                  axis=0, keepdims=True)
    eot_ref[...] = jnp.minimum(eot, float(e - 1)).astype(jnp.int32)

    er_col = jax.lax.broadcasted_iota(jnp.int32, (e, 1), 0)
    live = jnp.sum(jnp.where(er_col == e - 1, pad_end_col, 0.0),
                   axis=0, keepdims=True)
    live_ref[...] = (live / float(_TILE)).astype(jnp.int32)

    cpad_row = ((counts_ref[...].astype(jnp.int32) + _TILE - 1)
                // _TILE * _TILE).astype(jnp.float32)
    upm = jnp.where(er < ec, 1.0, 0.0).astype(jnp.bfloat16)
    pad_off_row = jax.lax.dot_general(
        cpad_row.astype(jnp.bfloat16), upm, (((1,), (0,)), ((), ())),
        preferred_element_type=jnp.float32)
    pos = pad_off_row + ranks
    p1 = jnp.sum(jnp.where(sel1, pos, 0.0), axis=1, keepdims=True)
    p2 = jnp.sum(jnp.where(sel2, pos, 0.0), axis=1, keepdims=True)

    metac = jnp.concatenate([p1, p2, w1, w2], axis=1)
    ident = jnp.where(ri == ci, 1.0, 0.0)
    metar_ref[...] = jax.lax.dot_general(
        metac, ident, (((0,), (0,)), ((), ())),
        precision=jax.lax.Precision.HIGHEST,
        preferred_element_type=jnp.float32)


def _routing(x, w_router, n_tiles):
    return pl.pallas_call(
        _routing_kernel,
        out_shape=(
            jax.ShapeDtypeStruct((1, _E), jnp.float32),
            jax.ShapeDtypeStruct((4, _T), jnp.float32),
            jax.ShapeDtypeStruct((1, n_tiles), jnp.int32),
            jax.ShapeDtypeStruct((1, 1), jnp.int32),
        ),
    )(x, w_router)


def _grouped_kernel(eot_ref, live_ref, xb_ref, metar_ref, win_ref, wout_ref,
                    o_ref, acc_ref):
    i = pl.program_id(0)
    n = pl.num_programs(0)

    @pl.when(i < live_ref[0])
    def _():
        r0 = i * _TILE
        rows = jax.lax.broadcasted_iota(jnp.int32, (_TILE, _T), 0) + r0
        p1 = metar_ref[0:1, :].astype(jnp.int32)
        p2 = metar_ref[1:2, :].astype(jnp.int32)
        m1 = jnp.where(rows == p1, 1.0, 0.0)
        m2 = jnp.where(rows == p2, 1.0, 0.0)
        disp = (m1 + m2).astype(jnp.bfloat16)
        xg = jax.lax.dot_general(
            disp, xb_ref[...], (((1,), (0,)), ((), ())),
            preferred_element_type=jnp.float32)
        h = jax.lax.dot_general(
            xg.astype(jnp.bfloat16), win_ref[0], (((1,), (0,)), ((), ())),
            preferred_element_type=jnp.float32)
        h = h * jax.nn.sigmoid(h)
        y = jax.lax.dot_general(
            h.astype(jnp.bfloat16), wout_ref[0], (((1,), (0,)), ((), ())),
            preferred_element_type=jnp.float32)
        ws = jnp.sum(m1 * metar_ref[2:3, :] + m2 * metar_ref[3:4, :],
                     axis=1, keepdims=True)
        yw = (y * ws).astype(jnp.bfloat16)
        nblk = _T // _TILE
        for k in range(nblk):
            contrib = jax.lax.dot_general(
                disp[:, k * _TILE:(k + 1) * _TILE], yw,
                (((0,), (0,)), ((), ())),
                preferred_element_type=jnp.float32)
            blk = pl.ds(k * _TILE, _TILE)

            @pl.when(i == 0)
            def _():
                acc_ref[blk, :] = contrib

            @pl.when(i != 0)
            def _():
                acc_ref[blk, :] = acc_ref[blk, :] + contrib

    @pl.when(i == n - 1)
    def _():
        o_ref[...] = acc_ref[...]


def _grouped_ffn(xb, metar, winb, woutb, eot, live):
    grid_spec = pltpu.PrefetchScalarGridSpec(
        num_scalar_prefetch=2,
        grid=(_NTILES,),
        in_specs=[
            pl.BlockSpec((_T, _D), lambda i, eot, live: (0, 0)),
            pl.BlockSpec((4, _T), lambda i, eot, live: (0, 0)),
            pl.BlockSpec((1, _D, _H), lambda i, eot, live: (eot[i], 0, 0)),
            pl.BlockSpec((1, _H, _D), lambda i, eot, live: (eot[i], 0, 0)),
        ],
        out_specs=pl.BlockSpec((_T, _D), lambda i, eot, live: (0, 0)),
        scratch_shapes=[pltpu.VMEM((_T, _D), jnp.float32)],
    )
    return pl.pallas_call(
        _grouped_kernel,
        grid_spec=grid_spec,
        out_shape=jax.ShapeDtypeStruct((_T, _D), jnp.float32),
        compiler_params=pltpu.CompilerParams(
            dimension_semantics=("arbitrary",)),
    )(eot, live, xb, metar, winb, woutb)


def kernel(x, w_router, w_in, w_out):
    counts, metar, eot, live = _routing(x, w_router, _NTILES)

    xb = x.astype(jnp.bfloat16)
    winb = w_in.astype(jnp.bfloat16)
    woutb = w_out.astype(jnp.bfloat16)

    out = _grouped_ffn(xb, metar, winb, woutb,
                       eot.reshape(_NTILES), live.reshape(1))
    return out, counts.reshape(_E)


# speedup vs baseline: 1.1554x; 1.1554x over previous
import jax
import jax.numpy as jnp
from jax.experimental import pallas as pl
from jax.experimental.pallas import tpu as pltpu

_D = 1024
_E = 8
_H = 2048
_T = 2048

_TILE = 256
_NTILES = (2 * _T + _E * (_TILE - 1) + _TILE - 1) // _TILE
_NPAD = _NTILES * _TILE


def _routing_kernel(x_ref, wr_ref, counts_ref, metar_ref, metac_ref,
                    eot_ref, live_ref):
    logits = jax.lax.dot_general(
        x_ref[...].astype(jnp.bfloat16), wr_ref[...].astype(jnp.bfloat16),
        (((1,), (0,)), ((), ())),
        preferred_element_type=jnp.float32)
    t, e = logits.shape
    col = jax.lax.broadcasted_iota(jnp.int32, (t, e), 1)

    v1 = jnp.max(logits, axis=1, keepdims=True)
    i1 = jnp.min(jnp.where(logits == v1, col, e), axis=1, keepdims=True)
    masked = jnp.where(col == i1, -jnp.inf, logits)
    v2 = jnp.max(masked, axis=1, keepdims=True)
    i2 = jnp.min(jnp.where(masked == v2, col, e), axis=1, keepdims=True)

    e2 = jnp.exp(v2 - v1)
    w1 = 1.0 / (1.0 + e2)
    w2 = 1.0 - w1

    sel1 = (col == i1)
    sel2 = (col == i2)
    onehot = jnp.where(sel1 | sel2, 1.0, 0.0)
    counts_ref[...] = jnp.sum(onehot, axis=0, keepdims=True)

    ri = jax.lax.broadcasted_iota(jnp.int32, (t, t), 0)
    ci = jax.lax.broadcasted_iota(jnp.int32, (t, t), 1)
    tri = jnp.where(ci < ri, 1.0, 0.0).astype(jnp.bfloat16)
    ranks = jax.lax.dot_general(
        tri, onehot.astype(jnp.bfloat16), (((1,), (0,)), ((), ())),
        preferred_element_type=jnp.float32)

    ones_col = jnp.full((t, 1), 1.0, dtype=jnp.bfloat16)
    counts_col = jax.lax.dot_general(
        onehot.astype(jnp.bfloat16), ones_col, (((0,), (0,)), ((), ())),
        preferred_element_type=jnp.float32)
    cpad_col = ((counts_col.astype(jnp.int32) + _TILE - 1)
                // _TILE * _TILE).astype(jnp.float32)

    er = jax.lax.broadcasted_iota(jnp.int32, (e, e), 0)
    ec = jax.lax.broadcasted_iota(jnp.int32, (e, e), 1)
    low = jnp.where(ec < er, 1.0, 0.0).astype(jnp.bfloat16)
    pad_off_col = jax.lax.dot_general(
        low, cpad_col.astype(jnp.bfloat16), (((1,), (0,)), ((), ())),
        preferred_element_type=jnp.float32)
    pad_end_col = pad_off_col + cpad_col

    ntiles = eot_ref.shape[1]
    tcol = (jax.lax.broadcasted_iota(jnp.int32, (e, ntiles), 1)
            * _TILE).astype(jnp.float32)
    eot = jnp.sum(jnp.where(pad_end_col <= tcol, 1.0, 0.0),
                  axis=0, keepdims=True)
    eot_ref[...] = jnp.minimum(eot, float(e - 1)).astype(jnp.int32)

    er_col = jax.lax.broadcasted_iota(jnp.int32, (e, 1), 0)
    live = jnp.sum(jnp.where(er_col == e - 1, pad_end_col, 0.0),
                   axis=0, keepdims=True)
    live_ref[...] = (live / float(_TILE)).astype(jnp.int32)

    cpad_row = ((counts_ref[...].astype(jnp.int32) + _TILE - 1)
                // _TILE * _TILE).astype(jnp.float32)
    upm = jnp.where(er < ec, 1.0, 0.0).astype(jnp.bfloat16)
    pad_off_row = jax.lax.dot_general(
        cpad_row.astype(jnp.bfloat16), upm, (((1,), (0,)), ((), ())),
        preferred_element_type=jnp.float32)
    pos = pad_off_row + ranks
    p1 = jnp.sum(jnp.where(sel1, pos, 0.0), axis=1, keepdims=True)
    p2 = jnp.sum(jnp.where(sel2, pos, 0.0), axis=1, keepdims=True)

    metac = jnp.concatenate([p1, p2, w1, w2], axis=1)
    metac_ref[...] = metac
    ident = jnp.where(ri == ci, 1.0, 0.0)
    metar_ref[...] = jax.lax.dot_general(
        metac, ident, (((0,), (0,)), ((), ())),
        precision=jax.lax.Precision.HIGHEST,
        preferred_element_type=jnp.float32)


def _routing(x, w_router, n_tiles):
    return pl.pallas_call(
        _routing_kernel,
        out_shape=(
            jax.ShapeDtypeStruct((1, _E), jnp.float32),
            jax.ShapeDtypeStruct((4, _T), jnp.float32),
            jax.ShapeDtypeStruct((_T, 4), jnp.float32),
            jax.ShapeDtypeStruct((1, n_tiles), jnp.int32),
            jax.ShapeDtypeStruct((1, 1), jnp.int32),
        ),
    )(x, w_router)


def _grouped_kernel(eot_ref, live_ref, xb_ref, metar_ref, win_ref, wout_ref,
                    ys_ref):
    i = pl.program_id(0) * (_NTILES // 2) + pl.program_id(1)

    @pl.when(i < live_ref[0])
    def _():
        r0 = i * _TILE
        rows = jax.lax.broadcasted_iota(jnp.int32, (_TILE, _T), 0) + r0
        p1 = metar_ref[0:1, :].astype(jnp.int32)
        p2 = metar_ref[1:2, :].astype(jnp.int32)
        disp = (jnp.where(rows == p1, 1.0, 0.0)
                + jnp.where(rows == p2, 1.0, 0.0)).astype(jnp.bfloat16)
        xg = jax.lax.dot_general(
            disp, xb_ref[...], (((1,), (0,)), ((), ())),
            preferred_element_type=jnp.float32)
        h = jax.lax.dot_general(
            xg.astype(jnp.bfloat16), win_ref[0], (((1,), (0,)), ((), ())),
            preferred_element_type=jnp.float32)
        h = h * jax.nn.sigmoid(h)
        y = jax.lax.dot_general(
            h.astype(jnp.bfloat16), wout_ref[0], (((1,), (0,)), ((), ())),
            preferred_element_type=jnp.float32)
        ys_ref[...] = y.astype(jnp.bfloat16)

    @pl.when(i >= live_ref[0])
    def _():
        ys_ref[...] = jnp.zeros(ys_ref.shape, jnp.bfloat16)


def _grouped_ffn(xb, metar, winb, woutb, eot, live):
    half = _NTILES // 2
    grid_spec = pltpu.PrefetchScalarGridSpec(
        num_scalar_prefetch=2,
        grid=(2, half),
        in_specs=[
            pl.BlockSpec((_T, _D), lambda c, i, eot, live: (0, 0)),
            pl.BlockSpec((4, _T), lambda c, i, eot, live: (0, 0)),
            pl.BlockSpec((1, _D, _H),
                         lambda c, i, eot, live: (eot[c * half + i], 0, 0)),
            pl.BlockSpec((1, _H, _D),
                         lambda c, i, eot, live: (eot[c * half + i], 0, 0)),
        ],
        out_specs=pl.BlockSpec((_TILE, _D),
                               lambda c, i, eot, live: (c * half + i, 0)),
    )
    return pl.pallas_call(
        _grouped_kernel,
        grid_spec=grid_spec,
        out_shape=jax.ShapeDtypeStruct((_NPAD, _D), jnp.bfloat16),
        compiler_params=pltpu.CompilerParams(
            dimension_semantics=("parallel", "arbitrary")),
    )(eot, live, xb, metar, winb, woutb)


def _combine_kernel(metac_ref, ys_ref, o_ref):
    lanes = jax.lax.broadcasted_iota(jnp.int32, (_TILE, _NPAD), 1)
    p1 = metac_ref[:, 0:1].astype(jnp.int32)
    p2 = metac_ref[:, 1:2].astype(jnp.int32)
    w1 = metac_ref[:, 2:3]
    w2 = metac_ref[:, 3:4]
    cmb = (jnp.where(lanes == p1, w1, 0.0)
           + jnp.where(lanes == p2, w2, 0.0)).astype(jnp.bfloat16)
    o_ref[...] = jax.lax.dot_general(
        cmb, ys_ref[...], (((1,), (0,)), ((), ())),
        preferred_element_type=jnp.float32)


def _combine(metac, ys):
    n_m = _T // _TILE
    half = n_m // 2
    return pl.pallas_call(
        _combine_kernel,
        grid=(2, half),
        in_specs=[
            pl.BlockSpec((_TILE, 4), lambda c, m: (c * half + m, 0)),
            pl.BlockSpec((_NPAD, _D), lambda c, m: (0, 0)),
        ],
        out_specs=pl.BlockSpec((_TILE, _D), lambda c, m: (c * half + m, 0)),
        out_shape=jax.ShapeDtypeStruct((_T, _D), jnp.float32),
        compiler_params=pltpu.CompilerParams(
            dimension_semantics=("parallel", "arbitrary")),
    )(metac, ys)


def kernel(x, w_router, w_in, w_out):
    counts, metar, metac, eot, live = _routing(x, w_router, _NTILES)

    xb = x.astype(jnp.bfloat16)
    winb = w_in.astype(jnp.bfloat16)
    woutb = w_out.astype(jnp.bfloat16)

    ys = _grouped_ffn(xb, metar, winb, woutb,
                      eot.reshape(_NTILES), live.reshape(1))
    out = _combine(metac, ys)
    return out, counts.reshape(_E)
